# baseline (speedup 1.0000x reference)
import functools

import jax
import jax.numpy as jnp
from jax import lax
from jax.experimental import pallas as pl
from jax.experimental.pallas import tpu as pltpu

D_MODEL = 512
NUM_TILES = 16
TOP_K = 4
B = 4096
TERNARY_W = 0.01
SPARSITY_W = 0.005

TBLK_A = 128
TBLK_B = 512
NEG_INF = -3.0e38


def _prep_body(ints_ref, op_embed_ref, w_in_ref, b_in_ref, w_r_ref, b_r_ref,
               x_ref, topi_ref, topn_ref, fg_ref, il_ref):
    i = pl.program_id(0)
    ints = ints_ref[...]
    op = ints[:, 0:1]
    a = ints[:, 1:2]
    b = ints[:, 2:3]
    c = ints[:, 3:4]
    T = ints.shape[0]

    op_emb = jnp.zeros((T, 16), jnp.float32)
    for j in range(8):
        m = (op == j).astype(jnp.float32)
        op_emb = op_emb + m * op_embed_ref[j:j + 1, :]

    bit_iota = lax.broadcasted_iota(jnp.int32, (1, 8), 1)
    a_bits = ((lax.shift_right_logical(a, bit_iota)) & 1).astype(jnp.float32)
    b_bits = ((lax.shift_right_logical(b, bit_iota)) & 1).astype(jnp.float32)
    c_f = c.astype(jnp.float32)

    feats = jnp.concatenate(
        [op_emb, a_bits, b_bits, c_f, jnp.zeros((T, 128 - 33), jnp.float32)],
        axis=1)

    x = jax.lax.dot_general(feats, w_in_ref[...], (((1,), (0,)), ((), ())),
                            preferred_element_type=jnp.float32)
    x = x + b_in_ref[...]
    x_ref[...] = x

    logits = jax.lax.dot_general(x, w_r_ref[...], (((1,), (0,)), ((), ())),
                                 preferred_element_type=jnp.float32)
    logits = logits + b_r_ref[...]
    m = jnp.max(logits, axis=1, keepdims=True)
    e = jnp.exp(logits - m)
    gates = e / jnp.sum(e, axis=1, keepdims=True)

    iota16 = lax.broadcasted_iota(jnp.int32, (T, NUM_TILES), 1)
    v = gates
    tis, tvs = [], []
    for _ in range(TOP_K):
        mx = jnp.max(v, axis=1, keepdims=True)
        is_mx = v == mx
        idx = jnp.min(jnp.where(is_mx, iota16, NUM_TILES), axis=1,
                      keepdims=True)
        tis.append(idx)
        tvs.append(mx)
        v = jnp.where(iota16 == idx, NEG_INF, v)
    topi = jnp.concatenate(tis, axis=1)
    topv = jnp.concatenate(tvs, axis=1)
    topn = topv / jnp.sum(topv, axis=1, keepdims=True)
    topi_ref[...] = topi
    topn_ref[...] = topn

    fg = jnp.zeros((T, NUM_TILES), jnp.float32)
    disp = jnp.zeros((T, NUM_TILES), jnp.float32)
    for k in range(TOP_K):
        sel = (iota16 == topi[:, k:k + 1]).astype(jnp.float32)
        fg = fg + sel * topn[:, k:k + 1]
        disp = disp + sel
    fg_ref[...] = fg

    @pl.when(i == 0)
    def _():
        il_ref[...] = jnp.zeros_like(il_ref)

    il_ref[0:1, 0:16] += jnp.sum(gates, axis=0, keepdims=True)
    il_ref[1:2, 0:16] += jnp.sum(disp, axis=0, keepdims=True)


def _prep_call(ints, op_embed, w_in_p, b_in, w_r, b_r):
    nblk = B // TBLK_A
    return pl.pallas_call(
        _prep_body,
        grid=(nblk,),
        in_specs=[
            pl.BlockSpec((TBLK_A, 4), lambda i: (i, 0)),
            pl.BlockSpec((8, 16), lambda i: (0, 0)),
            pl.BlockSpec((128, D_MODEL), lambda i: (0, 0)),
            pl.BlockSpec((1, D_MODEL), lambda i: (0, 0)),
            pl.BlockSpec((D_MODEL, NUM_TILES), lambda i: (0, 0)),
            pl.BlockSpec((1, NUM_TILES), lambda i: (0, 0)),
        ],
        out_specs=[
            pl.BlockSpec((TBLK_A, D_MODEL), lambda i: (i, 0)),
            pl.BlockSpec((TBLK_A, TOP_K), lambda i: (i, 0)),
            pl.BlockSpec((TBLK_A, TOP_K), lambda i: (i, 0)),
            pl.BlockSpec((TBLK_A, NUM_TILES), lambda i: (i, 0)),
            pl.BlockSpec((8, 128), lambda i: (0, 0)),
        ],
        out_shape=[
            jax.ShapeDtypeStruct((B, D_MODEL), jnp.float32),
            jax.ShapeDtypeStruct((B, TOP_K), jnp.int32),
            jax.ShapeDtypeStruct((B, TOP_K), jnp.float32),
            jax.ShapeDtypeStruct((B, NUM_TILES), jnp.float32),
            jax.ShapeDtypeStruct((8, 128), jnp.float32),
        ],
    )(ints, op_embed, w_in_p, b_in, w_r, b_r)


def _ffn_body(x_ref, fg_ref, w1_ref, b1_ref, w2_ref, b2_ref, out_ref):
    e = pl.program_id(1)
    T = x_ref.shape[0]

    @pl.when(e == 0)
    def _():
        out_ref[...] = jnp.zeros_like(out_ref)

    iota16 = lax.broadcasted_iota(jnp.int32, (T, NUM_TILES), 1)
    fge = jnp.sum(jnp.where(iota16 == e, fg_ref[...], 0.0), axis=1,
                  keepdims=True)
    h = jax.lax.dot_general(x_ref[...], w1_ref[0], (((1,), (0,)), ((), ())),
                            preferred_element_type=jnp.float32)
    h = jnp.maximum(h + b1_ref[0], 0.0)
    y = jax.lax.dot_general(h, w2_ref[0], (((1,), (0,)), ((), ())),
                            preferred_element_type=jnp.float32)
    y = y + b2_ref[0]
    out_ref[...] += fge * y


def _ffn_call(x, fg, w1, b1, w2, b2):
    nblk = B // TBLK_B
    return pl.pallas_call(
        _ffn_body,
        grid=(nblk, NUM_TILES),
        in_specs=[
            pl.BlockSpec((TBLK_B, D_MODEL), lambda t, e: (t, 0)),
            pl.BlockSpec((TBLK_B, NUM_TILES), lambda t, e: (t, 0)),
            pl.BlockSpec((1, D_MODEL, D_MODEL), lambda t, e: (e, 0, 0)),
            pl.BlockSpec((1, 1, D_MODEL), lambda t, e: (e, 0, 0)),
            pl.BlockSpec((1, D_MODEL, D_MODEL), lambda t, e: (e, 0, 0)),
            pl.BlockSpec((1, 1, D_MODEL), lambda t, e: (e, 0, 0)),
        ],
        out_specs=pl.BlockSpec((TBLK_B, D_MODEL), lambda t, e: (t, 0)),
        out_shape=jax.ShapeDtypeStruct((B, D_MODEL), jnp.float32),
    )(x, fg, w1, b1, w2, b2)


def _aux_body(w1_ref, w2_ref, il_ref, out_ref):
    e = pl.program_id(0)

    @pl.when(e == 0)
    def _():
        out_ref[...] = jnp.zeros_like(out_ref)

    aw1 = jnp.abs(w1_ref[0])
    aw2 = jnp.abs(w2_ref[0])
    s = (jnp.sum(aw1 * jnp.abs(1.0 - aw1)) + jnp.sum(aw2 * jnp.abs(1.0 - aw2)))
    out_ref[0:1, 0:1] += jnp.reshape(s, (1, 1))

    @pl.when(e == NUM_TILES - 1)
    def _():
        imp = il_ref[0:1, 0:16] * (1.0 / B)
        load = il_ref[1:2, 0:16] * (1.0 / B)
        lb = NUM_TILES * jnp.sum(imp * load)
        tern = out_ref[0, 0] / (NUM_TILES * D_MODEL * D_MODEL)
        out_ref[0:1, 0:1] = jnp.reshape(
            SPARSITY_W * lb + TERNARY_W * tern, (1, 1))


def _aux_call(w1, w2, il):
    return pl.pallas_call(
        _aux_body,
        grid=(NUM_TILES,),
        in_specs=[
            pl.BlockSpec((1, D_MODEL, D_MODEL), lambda e: (e, 0, 0)),
            pl.BlockSpec((1, D_MODEL, D_MODEL), lambda e: (e, 0, 0)),
            pl.BlockSpec((8, 128), lambda e: (0, 0)),
        ],
        out_specs=pl.BlockSpec((8, 128), lambda e: (0, 0)),
        out_shape=jax.ShapeDtypeStruct((8, 128), jnp.float32),
    )(w1, w2, il)


def _head_body(out_ref, wh1_ref, bh1_ref, wh2_ref, bh2_ref, rb_ref):
    h = jax.lax.dot_general(out_ref[...], wh1_ref[...],
                            (((1,), (0,)), ((), ())),
                            preferred_element_type=jnp.float32)
    h = jnp.maximum(h + bh1_ref[...], 0.0)
    z = jax.lax.dot_general(h, wh2_ref[...], (((1,), (0,)), ((), ())),
                            preferred_element_type=jnp.float32)
    z = z + bh2_ref[...]
    rb_ref[...] = 1.0 / (1.0 + jnp.exp(-z))


def _head_call(out, wh1_p, bh1_p, wh2_p, bh2):
    nblk = B // TBLK_B
    return pl.pallas_call(
        _head_body,
        grid=(nblk,),
        in_specs=[
            pl.BlockSpec((TBLK_B, D_MODEL), lambda i: (i, 0)),
            pl.BlockSpec((D_MODEL, 128), lambda i: (0, 0)),
            pl.BlockSpec((1, 128), lambda i: (0, 0)),
            pl.BlockSpec((128, 8), lambda i: (0, 0)),
            pl.BlockSpec((1, 8), lambda i: (0, 0)),
        ],
        out_specs=pl.BlockSpec((TBLK_B, 8), lambda i: (i, 0)),
        out_shape=jax.ShapeDtypeStruct((B, 8), jnp.float32),
    )(out, wh1_p, bh1_p, wh2_p, bh2)


def kernel(op_idx, a, b, c, op_embed, W_in, b_in, W_router, b_router,
           W1, b1, W2, b2, W_h1, b_h1, W_h2, b_h2):
    ints = jnp.stack([op_idx.astype(jnp.int32), a.astype(jnp.int32),
                      b.astype(jnp.int32), c.astype(jnp.int32)], axis=1)
    w_in_p = jnp.pad(W_in, ((0, 128 - 33), (0, 0)))
    wh1_p = jnp.pad(W_h1, ((0, 0), (0, 128 - 32)))
    bh1_p = jnp.pad(b_h1, (0, 128 - 32)).reshape(1, 128)
    wh2_p = jnp.pad(W_h2, ((0, 128 - 32), (0, 0)))

    x, topi, topn, fg, il = _prep_call(
        ints, op_embed, w_in_p, b_in.reshape(1, D_MODEL),
        W_router, b_router.reshape(1, NUM_TILES))
    out = _ffn_call(x, fg, W1, b1.reshape(NUM_TILES, 1, D_MODEL),
                    W2, b2.reshape(NUM_TILES, 1, D_MODEL))
    auxm = _aux_call(W1, W2, il)
    aux = auxm[0, 0]
    result_bits = _head_call(out, wh1_p, bh1_p, wh2_p, b_h2.reshape(1, 8))
    return result_bits, topi, aux
